# Initial kernel scaffold; baseline (speedup 1.0000x reference)
import jax
import jax.numpy as jnp
from jax.experimental import pallas as pl


def kernel(sample, tables, W, b):
    raise NotImplementedError("write your pallas kernel here")


# speedup vs baseline: 26.6052x; 26.6052x over previous
import math

import jax
import jax.numpy as jnp
from jax.experimental import pallas as pl
from jax.experimental.pallas import tpu as pltpu
from jax.experimental.pallas import tpu_sc as plsc

N_FEATURES = 8
VOCAB = 100000
D_EMBED = 16
D_MODEL = 128

SC_CORES = 2
SC_SUBCORES = 16

GATHER_WINDOW = 128
MM_BLOCK = 2048


def _sc_gather(table, flat_idx):
    n = flat_idx.shape[1]
    mesh = plsc.VectorSubcoreMesh(
        core_axis_name="core", subcore_axis_name="subcore"
    )

    @pl.kernel(
        out_type=jax.ShapeDtypeStruct((n, D_EMBED), jnp.float32),
        mesh=mesh,
        compiler_params=pltpu.CompilerParams(use_tc_tiling_on_sc=False),
    )
    def gather_kernel(tab_hbm, idx_hbm, out_hbm):
        def body(idx_v, out_v):
            pltpu.sync_copy(tab_hbm.at[idx_v.at[0]], out_v)

        pltpu.emit_pipeline(
            body,
            grid=(n // GATHER_WINDOW,),
            in_specs=[pl.BlockSpec((1, GATHER_WINDOW), lambda i: (0, i))],
            out_specs=[pl.BlockSpec((GATHER_WINDOW, D_EMBED), lambda i: (i, 0))],
            core_axis_name=("core", "subcore"),
            dimension_semantics=(pltpu.PARALLEL,),
        )(idx_hbm, out_hbm)

    return gather_kernel(table, flat_idx)


def _project(x, wt_scaled, b_scaled):
    m = x.shape[0]

    def body(x_ref, w_ref, b_ref, o_ref):
        o_ref[...] = (
            jnp.dot(x_ref[...], w_ref[...], preferred_element_type=jnp.float32)
            + b_ref[...]
        )

    return pl.pallas_call(
        body,
        grid=(m // MM_BLOCK,),
        in_specs=[
            pl.BlockSpec((MM_BLOCK, D_MODEL), lambda i: (i, 0)),
            pl.BlockSpec((D_MODEL, D_MODEL), lambda i: (0, 0)),
            pl.BlockSpec((1, D_MODEL), lambda i: (0, 0)),
        ],
        out_specs=pl.BlockSpec((MM_BLOCK, D_MODEL), lambda i: (i, 0)),
        out_shape=jax.ShapeDtypeStruct((m, D_MODEL), jnp.float32),
    )(x, wt_scaled, b_scaled)


def kernel(sample, tables, W, b):
    batch, seq, nf = sample.shape
    offs = jnp.arange(nf, dtype=jnp.int32) * VOCAB
    flat_idx = (sample + offs).reshape(1, -1)
    table = tables.reshape(nf * VOCAB, D_EMBED)

    emb = _sc_gather(table, flat_idx)
    x = emb.reshape(-1, nf * D_EMBED)

    scale = math.sqrt(D_MODEL)
    out = _project(x, W.T * scale, (b * scale).reshape(1, D_MODEL))
    return out.reshape(batch, seq, D_MODEL)


# speedup vs baseline: 26.6280x; 1.0009x over previous
import math

import jax
import jax.numpy as jnp
from jax.experimental import pallas as pl
from jax.experimental.pallas import tpu as pltpu
from jax.experimental.pallas import tpu_sc as plsc

N_FEATURES = 8
VOCAB = 100000
D_EMBED = 16
D_MODEL = 128

SC_CORES = 2
SC_SUBCORES = 16

GATHER_WINDOW = 128
MM_BLOCK = 2048


def _sc_gather(table, flat_idx):
    n = flat_idx.shape[1]
    mesh = plsc.VectorSubcoreMesh(
        core_axis_name="core", subcore_axis_name="subcore"
    )

    @pl.kernel(
        out_type=jax.ShapeDtypeStruct((n, D_EMBED), jnp.float32),
        mesh=mesh,
        compiler_params=pltpu.CompilerParams(use_tc_tiling_on_sc=False),
    )
    def gather_kernel(tab_hbm, idx_hbm, out_hbm):
        def body(idx_v, out_v):
            pltpu.sync_copy(tab_hbm.at[idx_v.at[0]], out_v)

        pltpu.emit_pipeline(
            body,
            grid=(n // GATHER_WINDOW,),
            in_specs=[pl.BlockSpec((1, GATHER_WINDOW), lambda i: (0, i))],
            out_specs=[pl.BlockSpec((GATHER_WINDOW, D_EMBED), lambda i: (i, 0))],
            core_axis_name=("core", "subcore"),
            dimension_semantics=(pltpu.PARALLEL,),
        )(idx_hbm, out_hbm)

    return gather_kernel(table, flat_idx)


def _project(x, wt_scaled, b_scaled):
    m = x.shape[0]

    def body(x_ref, w_ref, b_ref, o_ref):
        x16 = x_ref[...].astype(jnp.bfloat16)
        o_ref[...] = (
            jnp.dot(x16, w_ref[...], preferred_element_type=jnp.float32)
            + b_ref[...]
        )

    return pl.pallas_call(
        body,
        grid=(m // MM_BLOCK,),
        in_specs=[
            pl.BlockSpec((MM_BLOCK, D_MODEL), lambda i: (i, 0)),
            pl.BlockSpec((D_MODEL, D_MODEL), lambda i: (0, 0)),
            pl.BlockSpec((1, D_MODEL), lambda i: (0, 0)),
        ],
        out_specs=pl.BlockSpec((MM_BLOCK, D_MODEL), lambda i: (i, 0)),
        out_shape=jax.ShapeDtypeStruct((m, D_MODEL), jnp.float32),
    )(x, wt_scaled, b_scaled)


def kernel(sample, tables, W, b):
    batch, seq, nf = sample.shape
    offs = jnp.arange(nf, dtype=jnp.int32) * VOCAB
    flat_idx = (sample + offs).reshape(1, -1)
    table = tables.reshape(nf * VOCAB, D_EMBED)

    emb = _sc_gather(table, flat_idx)
    x = emb.reshape(-1, nf * D_EMBED)

    scale = math.sqrt(D_MODEL)
    wt = (W.T * scale).astype(jnp.bfloat16)
    out = _project(x, wt, (b * scale).reshape(1, D_MODEL))
    return out.reshape(batch, seq, D_MODEL)
